# Initial kernel scaffold; baseline (speedup 1.0000x reference)
import jax
import jax.numpy as jnp
from jax.experimental import pallas as pl


def kernel(x, batch, bias):
    raise NotImplementedError("write your pallas kernel here")


# speedup vs baseline: 4.2854x; 4.2854x over previous
import jax
import jax.numpy as jnp
from jax import lax
from jax.experimental import pallas as pl
from jax.experimental.pallas import tpu as pltpu
from jax.experimental.pallas import tpu_sc as plsc

_N = 100000
_D = 128
_S = 512
_NC = 2
_NS = 16
_NW = _NC * _NS

_C = 128
_NFULL = _N // _C
_TAIL = _N - _NFULL * _C
_K = _NFULL // _NW
_REM = _NFULL - _K * _NW
_ROWS = _S // _NS


def _seg_body(x_hbm, b_hbm, out_hbm, xbuf, ibuf, xtail, itail, zbuf, acc):
    c = lax.axis_index("c")
    s = lax.axis_index("s")
    wid = s * _NC + c

    zrow = jnp.zeros((16,), jnp.float32)
    for r in range(_ROWS):
        for f in range(_D // 16):
            zbuf[r, pl.ds(f * 16, 16)] = zrow
    pltpu.sync_copy(zbuf, acc.at[pl.ds(s * _ROWS, _ROWS)])
    plsc.subcore_barrier()

    def step(k):
        cid = wid + k * _NW
        base = cid * _C
        pltpu.sync_copy(x_hbm.at[pl.ds(base, _C)], xbuf)
        pltpu.sync_copy(b_hbm.at[pl.ds(base, _C)], ibuf)
        pltpu.sync_copy(xbuf, acc.at[ibuf], add=True)

    for k in range(_K):
        step(k)

    @pl.when(wid < _REM)
    def _():
        step(_K)

    @pl.when(wid == _REM)
    def _():
        base = _NFULL * _C
        pltpu.sync_copy(x_hbm.at[pl.ds(base, _TAIL)], xtail)
        pltpu.sync_copy(b_hbm.at[pl.ds(base, _TAIL)], itail)
        pltpu.sync_copy(xtail, acc.at[itail], add=True)

    plsc.subcore_barrier()
    pltpu.sync_copy(acc.at[pl.ds(s * _ROWS, _ROWS)],
                    out_hbm.at[pl.ds(c * _S + s * _ROWS, _ROWS)])


_mesh = plsc.VectorSubcoreMesh(
    core_axis_name="c", subcore_axis_name="s",
    num_cores=_NC, num_subcores=_NS)

_seg_sum = pl.kernel(
    _seg_body,
    out_type=jax.ShapeDtypeStruct((_NC * _S, _D), jnp.float32),
    mesh=_mesh,
    scratch_types=[
        pltpu.VMEM((_C, _D), jnp.float32),
        pltpu.VMEM((_C,), jnp.int32),
        pltpu.VMEM((_TAIL, _D), jnp.float32),
        pltpu.VMEM((_TAIL,), jnp.int32),
        pltpu.VMEM((_ROWS, _D), jnp.float32),
        pltpu.VMEM_SHARED((_S, _D), jnp.float32),
    ],
)


def _combine_body(p_ref, b_ref, o_ref):
    o_ref[...] = p_ref[:_S, :] + p_ref[_S:, :] + b_ref[0]


def kernel(x, batch, bias):
    b32 = batch.astype(jnp.int32)
    partials = _seg_sum(x, b32)
    bias_v = jnp.asarray(bias, jnp.float32).reshape(1)
    return pl.pallas_call(
        _combine_body,
        out_shape=jax.ShapeDtypeStruct((_S, _D), jnp.float32),
        in_specs=[
            pl.BlockSpec(memory_space=pltpu.VMEM),
            pl.BlockSpec(memory_space=pltpu.SMEM),
        ],
        out_specs=pl.BlockSpec(memory_space=pltpu.VMEM),
    )(partials, bias_v)


# speedup vs baseline: 6.6033x; 1.5409x over previous
import jax
import jax.numpy as jnp
from jax import lax
from jax.experimental import pallas as pl
from jax.experimental.pallas import tpu as pltpu
from jax.experimental.pallas import tpu_sc as plsc

_N = 100000
_D = 128
_S = 512
_NC = 2
_NS = 16
_NW = _NC * _NS

_C = 128
_NFULL = _N // _C
_TAIL = _N - _NFULL * _C
_K = _NFULL // _NW
_REM = _NFULL - _K * _NW
_ROWS = _S // _NS


def _seg_body(x_hbm, b_hbm, out_hbm, xbuf, ibuf, xtail, itail, zbuf, acc,
              semx, semi, semsc):
    c = lax.axis_index("c")
    s = lax.axis_index("s")
    wid = s * _NC + c

    def start(k, b):
        cid = wid + k * _NW
        base = cid * _C
        hx = pltpu.async_copy(x_hbm.at[pl.ds(base, _C)], xbuf.at[b], semx.at[b])
        hi = pltpu.async_copy(b_hbm.at[pl.ds(base, _C)], ibuf.at[b], semi.at[b])
        return hx, hi

    hin = [None, None]
    hin[0] = start(0, 0)

    zrow = jnp.zeros((16,), jnp.float32)
    for r in range(_ROWS):
        for f in range(_D // 16):
            zbuf[r, pl.ds(f * 16, 16)] = zrow
    pltpu.sync_copy(zbuf, acc.at[pl.ds(s * _ROWS, _ROWS)])
    plsc.subcore_barrier()

    hs = [None, None]
    for k in range(_K):
        b = k & 1
        nb = 1 - b
        if k + 1 < _K:
            if hs[nb] is not None:
                hs[nb].wait()
            hin[nb] = start(k + 1, nb)
        hx, hi = hin[b]
        hx.wait()
        hi.wait()
        hs[b] = pltpu.async_copy(xbuf.at[b], acc.at[ibuf.at[b]], semsc.at[b],
                                 add=True)
    for b in range(2):
        if hs[b] is not None:
            hs[b].wait()

    @pl.when(wid < _REM)
    def _():
        cid = wid + _K * _NW
        base = cid * _C
        pltpu.sync_copy(x_hbm.at[pl.ds(base, _C)], xbuf.at[0])
        pltpu.sync_copy(b_hbm.at[pl.ds(base, _C)], ibuf.at[0])
        pltpu.sync_copy(xbuf.at[0], acc.at[ibuf.at[0]], add=True)

    @pl.when(wid == _REM)
    def _():
        base = _NFULL * _C
        pltpu.sync_copy(x_hbm.at[pl.ds(base, _TAIL)], xtail)
        pltpu.sync_copy(b_hbm.at[pl.ds(base, _TAIL)], itail)
        pltpu.sync_copy(xtail, acc.at[itail], add=True)

    plsc.subcore_barrier()
    pltpu.sync_copy(acc.at[pl.ds(s * _ROWS, _ROWS)],
                    out_hbm.at[pl.ds(c * _S + s * _ROWS, _ROWS)])


_mesh = plsc.VectorSubcoreMesh(
    core_axis_name="c", subcore_axis_name="s",
    num_cores=_NC, num_subcores=_NS)

_seg_sum = pl.kernel(
    _seg_body,
    out_type=jax.ShapeDtypeStruct((_NC * _S, _D), jnp.float32),
    mesh=_mesh,
    scratch_types=[
        pltpu.VMEM((2, _C, _D), jnp.float32),
        pltpu.VMEM((2, _C), jnp.int32),
        pltpu.VMEM((_TAIL, _D), jnp.float32),
        pltpu.VMEM((_TAIL,), jnp.int32),
        pltpu.VMEM((_ROWS, _D), jnp.float32),
        pltpu.VMEM_SHARED((_S, _D), jnp.float32),
        pltpu.SemaphoreType.DMA((2,)),
        pltpu.SemaphoreType.DMA((2,)),
        pltpu.SemaphoreType.DMA((2,)),
    ],
)


def _combine_body(p_ref, b_ref, o_ref):
    o_ref[...] = p_ref[:_S, :] + p_ref[_S:, :] + b_ref[0]


def kernel(x, batch, bias):
    b32 = batch.astype(jnp.int32)
    partials = _seg_sum(x, b32)
    bias_v = jnp.asarray(bias, jnp.float32).reshape(1)
    return pl.pallas_call(
        _combine_body,
        out_shape=jax.ShapeDtypeStruct((_S, _D), jnp.float32),
        in_specs=[
            pl.BlockSpec(memory_space=pltpu.VMEM),
            pl.BlockSpec(memory_space=pltpu.SMEM),
        ],
        out_specs=pl.BlockSpec(memory_space=pltpu.VMEM),
    )(partials, bias_v)
